# Initial kernel scaffold; baseline (speedup 1.0000x reference)
import jax
import jax.numpy as jnp
from jax.experimental import pallas as pl


def kernel(coarse_matching_scores, gt_patch_corr_indices, gt_patch_corr_overlaps):
    raise NotImplementedError("write your pallas kernel here")


# speedup vs baseline: 21.9409x; 21.9409x over previous
import jax
import jax.numpy as jnp
from jax import lax
from jax.experimental import pallas as pl
from jax.experimental.pallas import tpu as pltpu
from jax.experimental.pallas import tpu_sc as plsc

N = 8192
M = 8192
C = 16384
THRESH = 0.1

L = 16
NW = 16
CHUNK = C // NW
BAND = N // NW
GCH = 128
NG = CHUNK // GCH


def _sc_body(scores_hbm, rows_hbm, cols_hbm, vals_hbm,
             mv_out, s_out,
             r_v, c_v, v_v, lrm, lcm, band2d, band_red,
             rm_all, cm_all, rm_sh, cm_sh, rm_v, cm_v,
             flat_v, s_v, mv_v, sem):
    w = lax.axis_index("s")
    base = w * CHUNK

    pltpu.sync_copy(rows_hbm.at[pl.ds(base, CHUNK)], r_v)
    pltpu.sync_copy(cols_hbm.at[pl.ds(base, CHUNK)], c_v)
    pltpu.sync_copy(vals_hbm.at[pl.ds(base, CHUNK)], v_v)

    zeros = jnp.zeros((L,), jnp.float32)

    def zbody(i, _):
        lrm[pl.ds(i * L, L)] = zeros
        lcm[pl.ds(i * L, L)] = zeros
        return 0
    lax.fori_loop(0, N // L, zbody, 0)

    iota = lax.iota(jnp.int32, L)

    def smax(table, idx_ref):
        def sbody(i, _):
            iv = idx_ref[pl.ds(i * L, L)]
            vv = v_v[pl.ds(i * L, L)]
            k, v = plsc.sort_key_val(iv, vv)
            for d in (1, 2, 4, 8):
                src = jnp.maximum(iota - d, 0)
                ks = k.at[src].get(mode="promise_in_bounds")
                vs = v.at[src].get(mode="promise_in_bounds")
                same = (ks == k) & (iota >= d)
                v = jnp.where(same, jnp.maximum(v, vs), v)
            nxt = jnp.minimum(iota + 1, L - 1)
            kn = k.at[nxt].get(mode="promise_in_bounds")
            is_last = (k != kn) | (iota == L - 1)
            cur = plsc.load_gather(table, [k], mask=is_last)
            newv = jnp.maximum(v, cur)
            plsc.store_scatter(table, [k], newv, mask=is_last)
            return 0
        lax.fori_loop(0, CHUNK // L, sbody, 0)

    smax(lrm, r_v)
    smax(lcm, c_v)

    pltpu.sync_copy(lrm, rm_all.at[w])
    pltpu.sync_copy(lcm, cm_all.at[w])
    plsc.subcore_barrier()

    def reduce_band(all_sh, final_sh):
        for u in range(NW):
            pltpu.sync_copy(all_sh.at[u, pl.ds(w * BAND, BAND)], band2d.at[u])

        def rbody(j, _):
            acc = band2d[0, pl.ds(j * L, L)]
            for u in range(1, NW):
                acc = jnp.maximum(acc, band2d[u, pl.ds(j * L, L)])
            band_red[pl.ds(j * L, L)] = acc
            return 0
        lax.fori_loop(0, BAND // L, rbody, 0)
        pltpu.sync_copy(band_red, final_sh.at[pl.ds(w * BAND, BAND)])

    reduce_band(rm_all, rm_sh)
    reduce_band(cm_all, cm_sh)
    plsc.subcore_barrier()

    pltpu.sync_copy(rm_sh, rm_v)
    pltpu.sync_copy(cm_sh, cm_v)

    def selbody(j, _):
        rv = r_v[pl.ds(j * L, L)]
        cv = c_v[pl.ds(j * L, L)]
        vv = v_v[pl.ds(j * L, L)]
        rm = plsc.load_gather(rm_v, [rv])
        cm = plsc.load_gather(cm_v, [cv])
        sel = (vv > THRESH) & (vv == rm) & (vv == cm)
        mv_v[pl.ds(j * L, L)] = jnp.where(sel, vv, 0.0)
        flat_v[pl.ds(j * L, L)] = rv * M + cv
        return 0
    lax.fori_loop(0, CHUNK // L, selbody, 0)

    copies = []
    for k in range(NG):
        copies.append(pltpu.async_copy(
            scores_hbm.at[flat_v.at[pl.ds(k * GCH, GCH)]],
            s_v.at[pl.ds(k * GCH, GCH)], sem))
    for cp in copies:
        cp.wait()

    pltpu.sync_copy(mv_v, mv_out.at[pl.ds(base, CHUNK)])
    pltpu.sync_copy(s_v, s_out.at[pl.ds(base, CHUNK)])


def _sc_stage(scores_flat, rows, cols, vals):
    mesh = plsc.VectorSubcoreMesh(
        core_axis_name="c", subcore_axis_name="s", num_cores=1)
    f32 = jnp.float32
    run = pl.kernel(
        _sc_body,
        out_type=(jax.ShapeDtypeStruct((C,), f32),
                  jax.ShapeDtypeStruct((C,), f32)),
        mesh=mesh,
        compiler_params=pltpu.CompilerParams(needs_layout_passes=False),
        scratch_types=[
            pltpu.VMEM((CHUNK,), jnp.int32),
            pltpu.VMEM((CHUNK,), jnp.int32),
            pltpu.VMEM((CHUNK,), f32),
            pltpu.VMEM((N,), f32),
            pltpu.VMEM((M,), f32),
            pltpu.VMEM((NW, BAND), f32),
            pltpu.VMEM((BAND,), f32),
            pltpu.MemorySpace.VMEM_SHARED((NW, N), f32),
            pltpu.MemorySpace.VMEM_SHARED((NW, M), f32),
            pltpu.MemorySpace.VMEM_SHARED((N,), f32),
            pltpu.MemorySpace.VMEM_SHARED((M,), f32),
            pltpu.VMEM((N,), f32),
            pltpu.VMEM((M,), f32),
            pltpu.VMEM((CHUNK,), jnp.int32),
            pltpu.VMEM((CHUNK,), f32),
            pltpu.VMEM((CHUNK,), f32),
            pltpu.SemaphoreType.DMA,
        ],
    )
    return run(scores_flat, rows, cols, vals)


def _tc_loss_body(mv_ref, s_ref, out_ref):
    mv = mv_ref[...]
    s = s_ref[...]
    num = jnp.sum(-jnp.log(s + 1e-8) * mv)
    den = jnp.sum(mv)
    out_ref[0, 0] = num / den


def _tc_stage(mv, s):
    return pl.pallas_call(
        _tc_loss_body,
        out_shape=jax.ShapeDtypeStruct((1, 1), jnp.float32),
        in_specs=[pl.BlockSpec(memory_space=pltpu.VMEM),
                  pl.BlockSpec(memory_space=pltpu.VMEM)],
        out_specs=pl.BlockSpec(memory_space=pltpu.SMEM),
    )(mv, s)


def kernel(coarse_matching_scores, gt_patch_corr_indices, gt_patch_corr_overlaps):
    scores_flat = coarse_matching_scores.reshape(-1)
    rows = gt_patch_corr_indices[:, 0]
    cols = gt_patch_corr_indices[:, 1]
    mv, sg = _sc_stage(scores_flat, rows, cols, gt_patch_corr_overlaps)
    loss = _tc_stage(mv.reshape(128, 128), sg.reshape(128, 128))
    return loss[0, 0]


# speedup vs baseline: 108.2178x; 4.9322x over previous
import jax
import jax.numpy as jnp
from jax import lax
from jax.experimental import pallas as pl
from jax.experimental.pallas import tpu as pltpu
from jax.experimental.pallas import tpu_sc as plsc

N = 8192
M = 8192
C = 16384
THRESH = 0.1

L = 16
NW = 16
CHUNK = C // NW
BAND = N // NW
GCH = 128
NG = CHUNK // GCH


def _sc_body(scores_hbm, rows_hbm, cols_hbm, vals_hbm,
             mv_out, s_out,
             r_v, c_v, v_v, lrm, lcm, band2d, band_red,
             rm_all, cm_all, rm_sh, cm_sh, rm_v, cm_v,
             flat_v, lane_v, g_v, s_v, mv_v, sem):
    w = lax.axis_index("s")
    base = w * CHUNK
    slines = scores_hbm.reshape(N * M // 128, 128)

    pltpu.sync_copy(rows_hbm.at[pl.ds(base, CHUNK)], r_v)
    pltpu.sync_copy(cols_hbm.at[pl.ds(base, CHUNK)], c_v)
    pltpu.sync_copy(vals_hbm.at[pl.ds(base, CHUNK)], v_v)

    zeros = jnp.zeros((L,), jnp.float32)

    def zbody(i, _):
        lrm[pl.ds(i * L, L)] = zeros
        lcm[pl.ds(i * L, L)] = zeros
        return 0
    lax.fori_loop(0, N // L, zbody, 0)

    iota = lax.iota(jnp.int32, L)

    def smax(table, idx_ref):
        def sbody(i, _):
            iv = idx_ref[pl.ds(i * L, L)]
            vv = v_v[pl.ds(i * L, L)]
            k, v = plsc.sort_key_val(iv, vv)
            for d in (1, 2, 4, 8):
                src = jnp.maximum(iota - d, 0)
                ks = k.at[src].get(mode="promise_in_bounds")
                vs = v.at[src].get(mode="promise_in_bounds")
                same = (ks == k) & (iota >= d)
                v = jnp.where(same, jnp.maximum(v, vs), v)
            nxt = jnp.minimum(iota + 1, L - 1)
            kn = k.at[nxt].get(mode="promise_in_bounds")
            is_last = (k != kn) | (iota == L - 1)
            cur = plsc.load_gather(table, [k], mask=is_last)
            newv = jnp.maximum(v, cur)
            plsc.store_scatter(table, [k], newv, mask=is_last)
            return 0
        lax.fori_loop(0, CHUNK // L, sbody, 0)

    smax(lrm, r_v)
    smax(lcm, c_v)

    pltpu.sync_copy(lrm, rm_all.at[w])
    pltpu.sync_copy(lcm, cm_all.at[w])
    plsc.subcore_barrier()

    def reduce_band(all_sh, final_sh):
        for u in range(NW):
            pltpu.sync_copy(all_sh.at[u, pl.ds(w * BAND, BAND)], band2d.at[u])

        def rbody(j, _):
            acc = band2d[0, pl.ds(j * L, L)]
            for u in range(1, NW):
                acc = jnp.maximum(acc, band2d[u, pl.ds(j * L, L)])
            band_red[pl.ds(j * L, L)] = acc
            return 0
        lax.fori_loop(0, BAND // L, rbody, 0)
        pltpu.sync_copy(band_red, final_sh.at[pl.ds(w * BAND, BAND)])

    reduce_band(rm_all, rm_sh)
    reduce_band(cm_all, cm_sh)
    plsc.subcore_barrier()

    pltpu.sync_copy(rm_sh, rm_v)
    pltpu.sync_copy(cm_sh, cm_v)

    def selbody(j, _):
        rv = r_v[pl.ds(j * L, L)]
        cv = c_v[pl.ds(j * L, L)]
        vv = v_v[pl.ds(j * L, L)]
        rm = plsc.load_gather(rm_v, [rv])
        cm = plsc.load_gather(cm_v, [cv])
        sel = (vv > THRESH) & (vv == rm) & (vv == cm)
        mv_v[pl.ds(j * L, L)] = jnp.where(sel, vv, 0.0)
        flat_v[pl.ds(j * L, L)] = (
            ((rv >> 3) << 9) | ((cv >> 7) << 3) | (rv & 7))
        lane_v[pl.ds(j * L, L)] = cv & 127
        return 0
    lax.fori_loop(0, CHUNK // L, selbody, 0)

    iota16 = lax.iota(jnp.int32, L)
    cp_prev = pltpu.async_copy(
        slines.at[flat_v.at[pl.ds(0, GCH)]], g_v.at[0], sem)
    for k in range(NG):
        cp_prev.wait()
        if k + 1 < NG:
            cp_prev = pltpu.async_copy(
                slines.at[flat_v.at[pl.ds((k + 1) * GCH, GCH)]],
                g_v.at[(k + 1) % 2], sem)
        gbuf = g_v.at[k % 2]
        for j2 in range(GCH // L):
            ri = jnp.full((L,), j2 * L, jnp.int32) + iota16
            ci = lane_v[pl.ds(k * GCH + j2 * L, L)]
            s_v[pl.ds(k * GCH + j2 * L, L)] = plsc.load_gather(gbuf, [ri, ci])

    pltpu.sync_copy(mv_v, mv_out.at[pl.ds(base, CHUNK)])
    pltpu.sync_copy(s_v, s_out.at[pl.ds(base, CHUNK)])


def _sc_stage(scores_flat, rows, cols, vals):
    mesh = plsc.VectorSubcoreMesh(
        core_axis_name="c", subcore_axis_name="s", num_cores=1)
    f32 = jnp.float32
    run = pl.kernel(
        _sc_body,
        out_type=(jax.ShapeDtypeStruct((C,), f32),
                  jax.ShapeDtypeStruct((C,), f32)),
        mesh=mesh,
        compiler_params=pltpu.CompilerParams(
            needs_layout_passes=False, use_tc_tiling_on_sc=True),
        scratch_types=[
            pltpu.VMEM((CHUNK,), jnp.int32),
            pltpu.VMEM((CHUNK,), jnp.int32),
            pltpu.VMEM((CHUNK,), f32),
            pltpu.VMEM((N,), f32),
            pltpu.VMEM((M,), f32),
            pltpu.VMEM((NW, BAND), f32),
            pltpu.VMEM((BAND,), f32),
            pltpu.MemorySpace.VMEM_SHARED((NW, N), f32),
            pltpu.MemorySpace.VMEM_SHARED((NW, M), f32),
            pltpu.MemorySpace.VMEM_SHARED((N,), f32),
            pltpu.MemorySpace.VMEM_SHARED((M,), f32),
            pltpu.VMEM((N,), f32),
            pltpu.VMEM((M,), f32),
            pltpu.VMEM((CHUNK,), jnp.int32),
            pltpu.VMEM((CHUNK,), jnp.int32),
            pltpu.VMEM((2, GCH, 128), f32),
            pltpu.VMEM((CHUNK,), f32),
            pltpu.VMEM((CHUNK,), f32),
            pltpu.SemaphoreType.DMA,
        ],
    )
    return run(scores_flat, rows, cols, vals)


def _tc_loss_body(mv_ref, s_ref, out_ref):
    mv = mv_ref[...]
    s = s_ref[...]
    num = jnp.sum(-jnp.log(s + 1e-8) * mv)
    den = jnp.sum(mv)
    out_ref[0, 0] = num / den


def _tc_stage(mv, s):
    return pl.pallas_call(
        _tc_loss_body,
        out_shape=jax.ShapeDtypeStruct((1, 1), jnp.float32),
        in_specs=[pl.BlockSpec(memory_space=pltpu.VMEM),
                  pl.BlockSpec(memory_space=pltpu.VMEM)],
        out_specs=pl.BlockSpec(memory_space=pltpu.SMEM),
    )(mv, s)


def kernel(coarse_matching_scores, gt_patch_corr_indices, gt_patch_corr_overlaps):
    rows = gt_patch_corr_indices[:, 0]
    cols = gt_patch_corr_indices[:, 1]
    scores4d = coarse_matching_scores.reshape(
        N // 8, 8, M // 128, 128).transpose(0, 2, 1, 3)
    mv, sg = _sc_stage(scores4d, rows, cols, gt_patch_corr_overlaps)
    loss = _tc_stage(mv.reshape(128, 128), sg.reshape(128, 128))
    return loss[0, 0]


# speedup vs baseline: 160.1166x; 1.4796x over previous
import jax
import jax.numpy as jnp
from jax import lax
from jax.experimental import pallas as pl
from jax.experimental.pallas import tpu as pltpu
from jax.experimental.pallas import tpu_sc as plsc

N = 8192
M = 8192
C = 16384
THRESH = 0.1

L = 16
NW = 16
CHUNK = C // NW
BAND = N // NW
GCH = 128
NG = CHUNK // GCH
ZU = 8


def _sc_body(scores_hbm, rows_hbm, cols_hbm, vals_hbm,
             mv_out, s_out,
             r_v, c_v, v_v, lrm, lcm, band_r, band_c, band_red,
             rm_all, cm_all, rm_sh, cm_sh, rm_v, cm_v,
             flat_v, s_v, mv_v, sem_g, sem_d):
    w = lax.axis_index("s")
    base = w * CHUNK
    sflat = scores_hbm

    pltpu.sync_copy(rows_hbm.at[pl.ds(base, CHUNK)], r_v)
    pltpu.sync_copy(cols_hbm.at[pl.ds(base, CHUNK)], c_v)
    pltpu.sync_copy(vals_hbm.at[pl.ds(base, CHUNK)], v_v)

    def fbody(j, _):
        rv = r_v[pl.ds(j * L, L)]
        cv = c_v[pl.ds(j * L, L)]
        flat_v[pl.ds(j * L, L)] = (
            ((rv >> 3) << 16) | ((cv >> 7) << 10) | ((rv & 7) << 7) | (cv & 127))
        return 0
    lax.fori_loop(0, CHUNK // L, fbody, 0)

    gathers = [
        pltpu.async_copy(sflat.at[flat_v.at[pl.ds(k * GCH, GCH)]],
                         s_v.at[pl.ds(k * GCH, GCH)], sem_g)
        for k in range(NG)
    ]

    zeros = jnp.zeros((L,), jnp.float32)

    def zbody(i, _):
        for u in range(ZU):
            lrm[pl.ds((i * ZU + u) * L, L)] = zeros
            lcm[pl.ds((i * ZU + u) * L, L)] = zeros
        return 0
    lax.fori_loop(0, N // L // ZU, zbody, 0)

    iota = lax.iota(jnp.int32, L)

    def smax(table, idx_ref):
        def sbody(i, _):
            iv = idx_ref[pl.ds(i * L, L)]
            vv = v_v[pl.ds(i * L, L)]
            k, v = plsc.sort_key_val(iv, vv)
            for d in (1, 2, 4, 8):
                src = jnp.maximum(iota - d, 0)
                ks = k.at[src].get(mode="promise_in_bounds")
                vs = v.at[src].get(mode="promise_in_bounds")
                same = (ks == k) & (iota >= d)
                v = jnp.where(same, jnp.maximum(v, vs), v)
            nxt = jnp.minimum(iota + 1, L - 1)
            kn = k.at[nxt].get(mode="promise_in_bounds")
            is_last = (k != kn) | (iota == L - 1)
            cur = plsc.load_gather(table, [k], mask=is_last)
            newv = jnp.maximum(v, cur)
            plsc.store_scatter(table, [k], newv, mask=is_last)
            return 0
        lax.fori_loop(0, CHUNK // L, sbody, 0)

    smax(lrm, r_v)
    smax(lcm, c_v)

    pltpu.sync_copy(lrm, rm_all.at[w])
    pltpu.sync_copy(lcm, cm_all.at[w])
    plsc.subcore_barrier()

    band_cps = []
    for u in range(NW):
        band_cps.append(pltpu.async_copy(
            rm_all.at[u, pl.ds(w * BAND, BAND)], band_r.at[u], sem_d))
        band_cps.append(pltpu.async_copy(
            cm_all.at[u, pl.ds(w * BAND, BAND)], band_c.at[u], sem_d))
    for cp in band_cps:
        cp.wait()

    def reduce_band(band2d, final_sh):
        def rbody(j, _):
            acc = band2d[0, pl.ds(j * L, L)]
            for u in range(1, NW):
                acc = jnp.maximum(acc, band2d[u, pl.ds(j * L, L)])
            band_red[pl.ds(j * L, L)] = acc
            return 0
        lax.fori_loop(0, BAND // L, rbody, 0)
        pltpu.sync_copy(band_red, final_sh.at[pl.ds(w * BAND, BAND)])

    reduce_band(band_r, rm_sh)
    reduce_band(band_c, cm_sh)
    plsc.subcore_barrier()

    pltpu.sync_copy(rm_sh, rm_v)
    pltpu.sync_copy(cm_sh, cm_v)

    def selbody(j, _):
        rv = r_v[pl.ds(j * L, L)]
        cv = c_v[pl.ds(j * L, L)]
        vv = v_v[pl.ds(j * L, L)]
        rm = plsc.load_gather(rm_v, [rv])
        cm = plsc.load_gather(cm_v, [cv])
        sel = (vv > THRESH) & (vv == rm) & (vv == cm)
        mv_v[pl.ds(j * L, L)] = jnp.where(sel, vv, 0.0)
        return 0
    lax.fori_loop(0, CHUNK // L, selbody, 0)

    for cp in gathers:
        cp.wait()

    pltpu.sync_copy(mv_v, mv_out.at[pl.ds(base, CHUNK)])
    pltpu.sync_copy(s_v, s_out.at[pl.ds(base, CHUNK)])


def _sc_stage(scores4d, rows, cols, vals):
    mesh = plsc.VectorSubcoreMesh(
        core_axis_name="c", subcore_axis_name="s", num_cores=1)
    f32 = jnp.float32
    run = pl.kernel(
        _sc_body,
        out_type=(jax.ShapeDtypeStruct((C,), f32),
                  jax.ShapeDtypeStruct((C,), f32)),
        mesh=mesh,
        compiler_params=pltpu.CompilerParams(
            needs_layout_passes=False, use_tc_tiling_on_sc=True),
        scratch_types=[
            pltpu.VMEM((CHUNK,), jnp.int32),
            pltpu.VMEM((CHUNK,), jnp.int32),
            pltpu.VMEM((CHUNK,), f32),
            pltpu.VMEM((N,), f32),
            pltpu.VMEM((M,), f32),
            pltpu.VMEM((NW, BAND), f32),
            pltpu.VMEM((NW, BAND), f32),
            pltpu.VMEM((BAND,), f32),
            pltpu.MemorySpace.VMEM_SHARED((NW, N), f32),
            pltpu.MemorySpace.VMEM_SHARED((NW, M), f32),
            pltpu.MemorySpace.VMEM_SHARED((N,), f32),
            pltpu.MemorySpace.VMEM_SHARED((M,), f32),
            pltpu.VMEM((N,), f32),
            pltpu.VMEM((M,), f32),
            pltpu.VMEM((CHUNK,), jnp.int32),
            pltpu.VMEM((CHUNK,), f32),
            pltpu.VMEM((CHUNK,), f32),
            pltpu.SemaphoreType.DMA,
            pltpu.SemaphoreType.DMA,
        ],
    )
    return run(scores4d, rows, cols, vals)


def _tc_loss_body(mv_ref, s_ref, out_ref):
    mv = mv_ref[...]
    s = s_ref[...]
    num = jnp.sum(-jnp.log(s + 1e-8) * mv)
    den = jnp.sum(mv)
    out_ref[0, 0] = num / den


def _tc_stage(mv, s):
    return pl.pallas_call(
        _tc_loss_body,
        out_shape=jax.ShapeDtypeStruct((1, 1), jnp.float32),
        in_specs=[pl.BlockSpec(memory_space=pltpu.VMEM),
                  pl.BlockSpec(memory_space=pltpu.VMEM)],
        out_specs=pl.BlockSpec(memory_space=pltpu.SMEM),
    )(mv, s)


def kernel(coarse_matching_scores, gt_patch_corr_indices, gt_patch_corr_overlaps):
    rows = gt_patch_corr_indices[:, 0]
    cols = gt_patch_corr_indices[:, 1]
    scores_phys = coarse_matching_scores.reshape(
        N // 8, 8, M // 128, 128).transpose(0, 2, 1, 3).reshape(-1)
    mv, sg = _sc_stage(scores_phys, rows, cols, gt_patch_corr_overlaps)
    loss = _tc_stage(mv.reshape(128, 128), sg.reshape(128, 128))
    return loss[0, 0]
